# baseline (speedup 1.0000x reference)
import functools

import jax
import jax.numpy as jnp
from jax.experimental import pallas as pl
from jax.experimental.pallas import tpu as pltpu

NUM_AGENTS = 8
HIDDEN = 1024
NZ = 2048
BATCH = 16384
ROWS = 1024
NBT = BATCH // ROWS
RB = 512
NSPLIT = HIDDEN // RB
NBSTEPS = NUM_AGENTS * NSPLIT


def _fused_kernel(aidx_ref, b_ref, w_ref, z_ref, out_ref,
                  parts_ref, scale_ref):
    step = pl.program_id(0)

    @pl.when(step < NBSTEPS)
    def _reduce():
        parts_ref[pl.ds(step, 1), :] = (
            jnp.max(b_ref[0], axis=0).astype(jnp.float32)[None, :])

        @pl.when(step == NBSTEPS - 1)
        def _finalize():
            parts = parts_ref[...].reshape(NUM_AGENTS, NSPLIT, NZ)
            masks = (jnp.max(parts, axis=1) > 0).astype(jnp.float32)
            alpha = jnp.sum(masks, axis=0)
            aidx = aidx_ref[0]
            onehot = (jax.lax.broadcasted_iota(
                jnp.int32, (NUM_AGENTS, 1), 0) == aidx).astype(jnp.float32)
            mask_sel = jnp.sum(masks * onehot, axis=0)
            weights = jnp.zeros((NZ,), jnp.float32)
            for k in range(NUM_AGENTS + 1):
                weights = jnp.where(alpha == float(k), w_ref[0, k], weights)
            scale_ref[0, :] = mask_sel * weights

    @pl.when(step >= NBSTEPS)
    def _mul():
        out_ref[...] = z_ref[...] * scale_ref[...]


@functools.partial(jax.jit, static_argnames=())
def kernel(Z_hat, B, w, agent_idx):
    B3 = B.reshape(NBSTEPS, RB, NZ)
    w2 = w.reshape(1, NUM_AGENTS + 1)
    aidx = jnp.asarray(agent_idx, jnp.int32).reshape((1,))

    out = pl.pallas_call(
        _fused_kernel,
        grid_spec=pltpu.PrefetchScalarGridSpec(
            num_scalar_prefetch=1,
            grid=(NBSTEPS + NBT,),
            in_specs=[
                pl.BlockSpec(
                    (1, RB, NZ),
                    lambda s, aidx: (jnp.minimum(s, NBSTEPS - 1), 0, 0),
                ),
                pl.BlockSpec((1, NUM_AGENTS + 1), lambda s, aidx: (0, 0)),
                pl.BlockSpec(
                    (ROWS, NZ),
                    lambda s, aidx: (jnp.maximum(s - NBSTEPS, 0), 0),
                ),
            ],
            out_specs=pl.BlockSpec(
                (ROWS, NZ),
                lambda s, aidx: (jnp.maximum(s - NBSTEPS, 0), 0),
            ),
            scratch_shapes=[
                pltpu.VMEM((NBSTEPS, NZ), jnp.float32),
                pltpu.VMEM((1, NZ), jnp.float32),
            ],
        ),
        out_shape=jax.ShapeDtypeStruct((BATCH, NZ), jnp.float32),
    )(aidx, B3, w2, Z_hat)
    return out
